# baseline (device time: 16774 ns/iter reference)
import jax
import jax.numpy as jnp
from jax import lax
from jax.experimental import pallas as pl
from jax.experimental.pallas import tpu as pltpu

N_CHUNK = 16


def kernel(x):
    _, m, n = x.shape
    half = n // 2
    mhalf = m // 2
    rows = mhalf // N_CHUNK

    def body(
        x_ref,
        out_ref,
        xsend_f32,
        xadd_f32,
        zbuf_s,
        zbuf_r,
        xbuf_r,
        zs_sems,
        zr_sems,
        xs_sems,
        xr_sems,
        loc_sems,
    ):
        my_x = lax.axis_index("x")
        my_y = lax.axis_index("y")
        my_z = lax.axis_index("z")
        peer_z = 1 - my_z
        peer_x = 1 - my_x
        row0 = my_x * mhalf
        orow0 = peer_x * mhalf

        cp_send = pltpu.make_async_copy(
            x_ref.at[0, pl.ds(row0, mhalf), pl.ds(peer_z * half, half)],
            xsend_f32,
            loc_sems.at[0],
        )
        cp_send.start()
        cp_add = pltpu.make_async_copy(
            x_ref.at[0, :, pl.ds(my_z * half, half)],
            xadd_f32,
            loc_sems.at[1],
        )
        cp_add.start()

        barrier_sem = pltpu.get_barrier_semaphore()
        for dev in ((my_x, my_y, peer_z), (peer_x, my_y, my_z)):
            pl.semaphore_signal(
                barrier_sem,
                inc=1,
                device_id=dev,
                device_id_type=pl.DeviceIdType.MESH,
            )
        cp_send.wait()
        zbuf_s[...] = xsend_f32[...].astype(jnp.bfloat16)
        pl.semaphore_wait(barrier_sem, 2)

        z_rdmas = []
        for c in range(N_CHUNK):
            r = pltpu.make_async_remote_copy(
                src_ref=zbuf_s.at[pl.ds(c * rows, rows)],
                dst_ref=zbuf_r.at[pl.ds(c * rows, rows)],
                send_sem=zs_sems.at[c],
                recv_sem=zr_sems.at[c],
                device_id=(my_x, my_y, peer_z),
                device_id_type=pl.DeviceIdType.MESH,
            )
            r.start()
            z_rdmas.append(r)

        cp_add.wait()

        x_rdmas = []
        for c in range(N_CHUNK):
            z_rdmas[c].wait_recv()
            r = pltpu.make_async_remote_copy(
                src_ref=zbuf_r.at[pl.ds(c * rows, rows)],
                dst_ref=xbuf_r.at[pl.ds(c * rows, rows)],
                send_sem=xs_sems.at[c],
                recv_sem=xr_sems.at[c],
                device_id=(peer_x, my_y, my_z),
                device_id_type=pl.DeviceIdType.MESH,
            )
            r.start()
            x_rdmas.append(r)
            out_ref[pl.ds(row0 + c * rows, rows), :] = (
                xadd_f32[pl.ds(row0 + c * rows, rows), :]
                + zbuf_r[pl.ds(c * rows, rows), :].astype(jnp.float32)
            ).astype(jnp.bfloat16)

        for c in range(N_CHUNK):
            x_rdmas[c].wait_recv()
            out_ref[pl.ds(orow0 + c * rows, rows), :] = (
                xadd_f32[pl.ds(orow0 + c * rows, rows), :]
                + xbuf_r[pl.ds(c * rows, rows), :].astype(jnp.float32)
            ).astype(jnp.bfloat16)

        for c in range(N_CHUNK):
            z_rdmas[c].wait_send()
            x_rdmas[c].wait_send()

    return pl.pallas_call(
        body,
        out_shape=jax.ShapeDtypeStruct((m, half), jnp.bfloat16),
        in_specs=[pl.BlockSpec(memory_space=pltpu.MemorySpace.HBM)],
        out_specs=pl.BlockSpec(memory_space=pltpu.VMEM),
        scratch_shapes=[
            pltpu.VMEM((mhalf, half), jnp.float32),
            pltpu.VMEM((m, half), jnp.float32),
            pltpu.VMEM((mhalf, half), jnp.bfloat16),
            pltpu.VMEM((mhalf, half), jnp.bfloat16),
            pltpu.VMEM((mhalf, half), jnp.bfloat16),
            pltpu.SemaphoreType.DMA((N_CHUNK,)),
            pltpu.SemaphoreType.DMA((N_CHUNK,)),
            pltpu.SemaphoreType.DMA((N_CHUNK,)),
            pltpu.SemaphoreType.DMA((N_CHUNK,)),
            pltpu.SemaphoreType.DMA((2,)),
        ],
        compiler_params=pltpu.CompilerParams(collective_id=0),
    )(x)


# device time: 16125 ns/iter; 1.0402x vs baseline; 1.0402x over previous
import jax
import jax.numpy as jnp
from jax import lax
from jax.experimental import pallas as pl
from jax.experimental.pallas import tpu as pltpu

N_CHUNK = 16


def kernel(x):
    _, m, n = x.shape
    half = n // 2
    mhalf = m // 2
    rows = mhalf // N_CHUNK

    def body(
        x_ref,
        out_ref,
        zbuf_s,
        zbuf_r,
        xbuf_r,
        zs_sems,
        zr_sems,
        xs_sems,
        xr_sems,
    ):
        my_x = lax.axis_index("x")
        my_y = lax.axis_index("y")
        my_z = lax.axis_index("z")
        peer_z = 1 - my_z
        peer_x = 1 - my_x
        row0 = my_x * mhalf
        orow0 = peer_x * mhalf

        barrier_sem = pltpu.get_barrier_semaphore()
        for dev in ((my_x, my_y, peer_z), (peer_x, my_y, my_z)):
            pl.semaphore_signal(
                barrier_sem,
                inc=1,
                device_id=dev,
                device_id_type=pl.DeviceIdType.MESH,
            )
        zbuf_s[...] = x_ref[
            0, pl.ds(row0, mhalf), pl.ds(peer_z * half, half)
        ].astype(jnp.bfloat16)
        pl.semaphore_wait(barrier_sem, 2)

        z_rdmas = []
        for c in range(N_CHUNK):
            r = pltpu.make_async_remote_copy(
                src_ref=zbuf_s.at[pl.ds(c * rows, rows)],
                dst_ref=zbuf_r.at[pl.ds(c * rows, rows)],
                send_sem=zs_sems.at[c],
                recv_sem=zr_sems.at[c],
                device_id=(my_x, my_y, peer_z),
                device_id_type=pl.DeviceIdType.MESH,
            )
            r.start()
            z_rdmas.append(r)

        x_rdmas = []
        for c in range(N_CHUNK):
            z_rdmas[c].wait_recv()
            r = pltpu.make_async_remote_copy(
                src_ref=zbuf_r.at[pl.ds(c * rows, rows)],
                dst_ref=xbuf_r.at[pl.ds(c * rows, rows)],
                send_sem=xs_sems.at[c],
                recv_sem=xr_sems.at[c],
                device_id=(peer_x, my_y, my_z),
                device_id_type=pl.DeviceIdType.MESH,
            )
            r.start()
            x_rdmas.append(r)
            out_ref[pl.ds(row0 + c * rows, rows), :] = (
                x_ref[
                    0, pl.ds(row0 + c * rows, rows), pl.ds(my_z * half, half)
                ]
                + zbuf_r[pl.ds(c * rows, rows), :].astype(jnp.float32)
            ).astype(jnp.bfloat16)

        for c in range(N_CHUNK):
            x_rdmas[c].wait_recv()
            out_ref[pl.ds(orow0 + c * rows, rows), :] = (
                x_ref[
                    0, pl.ds(orow0 + c * rows, rows), pl.ds(my_z * half, half)
                ]
                + xbuf_r[pl.ds(c * rows, rows), :].astype(jnp.float32)
            ).astype(jnp.bfloat16)

        for c in range(N_CHUNK):
            z_rdmas[c].wait_send()
            x_rdmas[c].wait_send()

    return pl.pallas_call(
        body,
        out_shape=jax.ShapeDtypeStruct((m, half), jnp.bfloat16),
        in_specs=[pl.BlockSpec(memory_space=pltpu.VMEM)],
        out_specs=pl.BlockSpec(memory_space=pltpu.VMEM),
        scratch_shapes=[
            pltpu.VMEM((mhalf, half), jnp.bfloat16),
            pltpu.VMEM((mhalf, half), jnp.bfloat16),
            pltpu.VMEM((mhalf, half), jnp.bfloat16),
            pltpu.SemaphoreType.DMA((N_CHUNK,)),
            pltpu.SemaphoreType.DMA((N_CHUNK,)),
            pltpu.SemaphoreType.DMA((N_CHUNK,)),
            pltpu.SemaphoreType.DMA((N_CHUNK,)),
        ],
        compiler_params=pltpu.CompilerParams(collective_id=0),
    )(x)
